# baseline (speedup 1.0000x reference)
import functools

import jax
import jax.numpy as jnp
from jax import lax
from jax.experimental import pallas as pl
from jax.experimental.pallas import tpu as pltpu
from jax.experimental.pallas import tpu_sc as plsc

B = 16384
D = 32
L = 16
NC = 2
NS = 16
NW = NC * NS
BPW = B // NW
CHUNK = 128
NCHUNK = BPW // CHUNK

LN2 = 0.6931471805599453
LN_MAXV = 13.815510557964274
SQRT2 = 1.4142135623730951
SCALE = 0.75

_mesh = plsc.VectorSubcoreMesh(core_axis_name="c", subcore_axis_name="s")


def _ln(x):
    bits = lax.bitcast_convert_type(x, jnp.int32)
    e = lax.shift_right_logical(bits, 23) - 127
    m = lax.bitcast_convert_type(
        (bits & 0x007FFFFF) | 0x3F800000, jnp.float32)
    big = m > SQRT2
    m = jnp.where(big, m * 0.5, m)
    e = e + jnp.where(big, 1, 0)
    z = (m - 1.0) / (m + 1.0)
    z2 = z * z
    p = z * (2.0 + z2 * (2.0 / 3.0 + z2 * (2.0 / 5.0 + z2 * (2.0 / 7.0 + z2 * (2.0 / 9.0)))))
    return e.astype(jnp.float32) * LN2 + p


@functools.partial(
    pl.kernel,
    out_type=jax.ShapeDtypeStruct((NW, L), jnp.float32),
    mesh=_mesh,
    compiler_params=pltpu.CompilerParams(needs_layout_passes=False,
                                         use_tc_tiling_on_sc=False),
    scratch_types=[
        pltpu.VMEM((NCHUNK, CHUNK), jnp.int32),
        pltpu.VMEM((NCHUNK, CHUNK), jnp.int32),
        pltpu.VMEM((NCHUNK, CHUNK), jnp.int32),
        pltpu.VMEM((NCHUNK, CHUNK), jnp.int32),
        pltpu.VMEM((2, CHUNK, 128), jnp.float32),
        pltpu.VMEM((2, CHUNK, 128), jnp.float32),
        pltpu.VMEM((BPW,), jnp.float32),
        pltpu.VMEM((BPW,), jnp.float32),
        pltpu.VMEM((BPW,), jnp.float32),
        pltpu.VMEM((L * (L + 1),), jnp.float32),
        pltpu.VMEM((L,), jnp.float32),
        pltpu.SemaphoreType.DMA,
        pltpu.SemaphoreType.DMA,
    ],
)
def _glove_cost(t_ind, c_ind, co_hbm, t_emb, c_emb, t_bias, c_bias, out,
                t_idx, c_idx, t_rid, c_rid, t_stage, c_stage, tb_v, cb_v,
                co_v, pad, acc_v, sem, sem2):
    wid = lax.axis_index("s") * NC + lax.axis_index("c")
    base = wid * BPW

    for c in range(NCHUNK):
        pltpu.sync_copy(t_ind.at[pl.ds(base + c * CHUNK, CHUNK)], t_idx.at[c])
        pltpu.sync_copy(c_ind.at[pl.ds(base + c * CHUNK, CHUNK)], c_idx.at[c])
    pltpu.sync_copy(co_hbm.at[pl.ds(base, BPW)], co_v)

    for c in range(NCHUNK):
        for v in range(CHUNK // L):
            sl = pl.ds(v * L, L)
            t_idx16 = t_idx[c, sl]
            c_idx16 = c_idx[c, sl]
            t_rid[c, sl] = lax.shift_right_logical(t_idx16, 2)
            c_rid[c, sl] = lax.shift_right_logical(c_idx16, 2)

    bias_copies = []
    for c in range(NCHUNK):
        sl = pl.ds(c * CHUNK, CHUNK)
        bias_copies.append(
            pltpu.async_copy(t_bias.at[t_idx.at[c]], tb_v.at[sl], sem2))
        bias_copies.append(
            pltpu.async_copy(c_bias.at[c_idx.at[c]], cb_v.at[sl], sem2))

    lanes17 = lax.broadcasted_iota(jnp.int32, (L,), 0) * (L + 1)

    def fire(c, buf):
        tc = pltpu.async_copy(t_emb.at[t_rid.at[c]], t_stage.at[buf], sem)
        cc = pltpu.async_copy(c_emb.at[c_rid.at[c]], c_stage.at[buf], sem)
        return tc, cc

    pending = fire(0, 0)
    for cp in bias_copies:
        cp.wait()

    acc = jnp.zeros((L,), jnp.float32)
    for c in range(NCHUNK):
        pending[0].wait()
        pending[1].wait()
        if c + 1 < NCHUNK:
            pending = fire(c + 1, (c + 1) % 2)
        buf = c % 2

        def group_body(g, acc, c=c, buf=buf):
            rbase = g * L
            tiv = t_idx[c, pl.ds(rbase, L)]
            civ = c_idx[c, pl.ds(rbase, L)]
            toffv = (tiv & 3) * D
            coffv = (civ & 3) * D
            for r in range(L):
                row = rbase + r
                toff = toffv[r]
                coff = coffv[r]
                a0 = t_stage[buf, row, pl.ds(toff, L)]
                a1 = t_stage[buf, row, pl.ds(toff + L, L)]
                b0 = c_stage[buf, row, pl.ds(coff, L)]
                b1 = c_stage[buf, row, pl.ds(coff + L, L)]
                pad[pl.ds(r * (L + 1), L)] = a0 * b0 + a1 * b1
            d = jnp.zeros((L,), jnp.float32)
            for j in range(L):
                d = d + plsc.load_gather(pad, [lanes17 + j])
            abase = c * CHUNK + rbase
            tb = tb_v[pl.ds(abase, L)]
            cb = cb_v[pl.ds(abase, L)]
            co = co_v[pl.ds(abase, L)]
            lnco = _ln(co)
            w = jnp.minimum(1.0, jnp.exp(SCALE * (lnco - LN_MAXV)))
            err = d + tb + cb - lnco
            return acc + w * err * err

        acc = lax.fori_loop(0, CHUNK // L, group_body, acc)

    acc_v[...] = acc
    pltpu.sync_copy(acc_v, out.at[wid])


def kernel(target_ind, context_ind, co_occurs, target_embeddings,
           context_embeddings, target_biases, context_biases):
    t2 = target_embeddings.reshape(-1, 128)
    c2 = context_embeddings.reshape(-1, 128)
    partials = _glove_cost(target_ind, context_ind, co_occurs,
                           t2, c2, target_biases, context_biases)
    return jnp.sum(partials)


# speedup vs baseline: 1.3726x; 1.3726x over previous
import functools

import jax
import jax.numpy as jnp
from jax import lax
from jax.experimental import pallas as pl
from jax.experimental.pallas import tpu as pltpu
from jax.experimental.pallas import tpu_sc as plsc

B = 16384
D = 32
L = 16
NC = 2
NS = 16
NW = NC * NS
BPW = B // NW
CHUNK = 128
NCHUNK = BPW // CHUNK
SUB = 32

LN2 = 0.6931471805599453
LN_MAXV = 13.815510557964274
SQRT2 = 1.4142135623730951
SCALE = 0.75

_mesh = plsc.VectorSubcoreMesh(core_axis_name="c", subcore_axis_name="s")


def _ln(x):
    bits = lax.bitcast_convert_type(x, jnp.int32)
    e = lax.shift_right_logical(bits, 23) - 127
    m = lax.bitcast_convert_type(
        (bits & 0x007FFFFF) | 0x3F800000, jnp.float32)
    big = m > SQRT2
    m = jnp.where(big, m * 0.5, m)
    e = e + jnp.where(big, 1, 0)
    z = (m - 1.0) / (m + 1.0)
    z2 = z * z
    p = z * (2.0 + z2 * (2.0 / 3.0 + z2 * (2.0 / 5.0 + z2 * (2.0 / 7.0 + z2 * (2.0 / 9.0)))))
    return e.astype(jnp.float32) * LN2 + p


@functools.partial(
    pl.kernel,
    out_type=jax.ShapeDtypeStruct((NW, L), jnp.float32),
    mesh=_mesh,
    compiler_params=pltpu.CompilerParams(needs_layout_passes=False,
                                         use_tc_tiling_on_sc=True),
    scratch_types=[
        pltpu.VMEM((NCHUNK, CHUNK), jnp.int32),
        pltpu.VMEM((NCHUNK, CHUNK), jnp.int32),
        pltpu.VMEM((SUB, 8, D), jnp.float32),
        pltpu.VMEM((SUB, 8, D), jnp.float32),
        pltpu.VMEM((BPW,), jnp.float32),
        pltpu.VMEM((BPW,), jnp.float32),
        pltpu.VMEM((BPW,), jnp.float32),
        pltpu.VMEM((L * (L + 1),), jnp.float32),
        pltpu.VMEM((L,), jnp.float32),
        pltpu.SemaphoreType.DMA,
        pltpu.SemaphoreType.DMA,
    ],
)
def _glove_cost(t_ind, c_ind, co_hbm, t_emb, c_emb, t_bias, c_bias, out,
                t_idx, c_idx, t_blk, c_blk, tb_v, cb_v, co_v, pad, acc_v,
                sem, sem2):
    wid = lax.axis_index("s") * NC + lax.axis_index("c")
    base = wid * BPW

    for c in range(NCHUNK):
        pltpu.sync_copy(t_ind.at[pl.ds(base + c * CHUNK, CHUNK)], t_idx.at[c])
        pltpu.sync_copy(c_ind.at[pl.ds(base + c * CHUNK, CHUNK)], c_idx.at[c])
    pltpu.sync_copy(co_hbm.at[pl.ds(base, BPW)], co_v)

    bias_copies = []
    for c in range(NCHUNK):
        sl = pl.ds(c * CHUNK, CHUNK)
        bias_copies.append(
            pltpu.async_copy(t_bias.at[t_idx.at[c]], tb_v.at[sl], sem2))
        bias_copies.append(
            pltpu.async_copy(c_bias.at[c_idx.at[c]], cb_v.at[sl], sem2))

    lanes17 = lax.broadcasted_iota(jnp.int32, (L,), 0) * (L + 1)

    def fire_sub(c, h):
        def fire_group(g, carry):
            off = h * SUB + g * L
            tiv = t_idx[c, pl.ds(off, L)]
            civ = c_idx[c, pl.ds(off, L)]
            tav = lax.shift_left(lax.shift_right_logical(tiv, 3), 3)
            cav = lax.shift_left(lax.shift_right_logical(civ, 3), 3)
            for r in range(L):
                row = g * L + r
                tstart = pl.multiple_of(tav[r], 8)
                cstart = pl.multiple_of(cav[r], 8)
                pltpu.async_copy(t_emb.at[pl.ds(tstart, 8)],
                                 t_blk.at[row], sem)
                pltpu.async_copy(c_emb.at[pl.ds(cstart, 8)],
                                 c_blk.at[row], sem)
            return carry

        lax.fori_loop(0, SUB // L, fire_group, 0)

    def drain_sub():
        def drain_group(g, carry):
            for _ in range(L):
                pltpu.make_async_copy(t_emb.at[pl.ds(0, 8)],
                                      t_blk.at[0], sem).wait()
                pltpu.make_async_copy(c_emb.at[pl.ds(0, 8)],
                                      c_blk.at[0], sem).wait()
            return carry

        lax.fori_loop(0, SUB // L, drain_group, 0)

    def compute_sub(c, h, acc):
        def group_body(g, acc):
            rbase = g * L
            off = h * SUB + rbase
            tiv = t_idx[c, pl.ds(off, L)]
            civ = c_idx[c, pl.ds(off, L)]
            tsv = tiv & 7
            csv = civ & 7
            for r in range(L):
                row = rbase + r
                ts = tsv[r]
                cs = csv[r]
                a0 = t_blk[row, ts, pl.ds(0, L)]
                a1 = t_blk[row, ts, pl.ds(L, L)]
                b0 = c_blk[row, cs, pl.ds(0, L)]
                b1 = c_blk[row, cs, pl.ds(L, L)]
                pad[pl.ds(r * (L + 1), L)] = a0 * b0 + a1 * b1
            d = jnp.zeros((L,), jnp.float32)
            for j in range(L):
                d = d + plsc.load_gather(pad, [lanes17 + j])
            abase = c * CHUNK + h * SUB + rbase
            tb = tb_v[pl.ds(abase, L)]
            cb = cb_v[pl.ds(abase, L)]
            co = co_v[pl.ds(abase, L)]
            lnco = _ln(co)
            w = jnp.minimum(1.0, jnp.exp(SCALE * (lnco - LN_MAXV)))
            err = d + tb + cb - lnco
            return acc + w * err * err

        return lax.fori_loop(0, SUB // L, group_body, acc)

    for cp in bias_copies:
        cp.wait()

    acc = jnp.zeros((L,), jnp.float32)
    for c in range(NCHUNK):
        for h in range(CHUNK // SUB):
            fire_sub(c, h)
            drain_sub()
            acc = compute_sub(c, h, acc)

    acc_v[...] = acc
    pltpu.sync_copy(acc_v, out.at[wid])


def kernel(target_ind, context_ind, co_occurs, target_embeddings,
           context_embeddings, target_biases, context_biases):
    partials = _glove_cost(target_ind, context_ind, co_occurs,
                           target_embeddings, context_embeddings,
                           target_biases, context_biases)
    return jnp.sum(partials)
